# baseline (speedup 1.0000x reference)
import functools

import jax
import jax.numpy as jnp
from jax import lax
from jax.experimental import pallas as pl
from jax.experimental.pallas import tpu as pltpu
from jax.experimental.pallas import tpu_sc as plsc

RB = 1000


def _mm_bias_body(x_ref, w_ref, b_ref, o_ref):
    o_ref[...] = (
        jnp.dot(x_ref[...], w_ref[...], preferred_element_type=jnp.float32)
        + b_ref[...]
    )


def _tc_mm_bias(x, w, b):
    n, din = x.shape
    h = w.shape[1]
    return pl.pallas_call(
        _mm_bias_body,
        grid=(n // RB,),
        in_specs=[
            pl.BlockSpec((RB, din), lambda i: (i, 0)),
            pl.BlockSpec((din, h), lambda i: (0, 0)),
            pl.BlockSpec((1, h), lambda i: (0, 0)),
        ],
        out_specs=pl.BlockSpec((RB, h), lambda i: (i, 0)),
        out_shape=jax.ShapeDtypeStruct((n, h), jnp.float32),
    )(x, w, b.reshape(1, h))


def _stage_a_body(h_ref, pe_ref, wc_ref, ws_ref, bk_ref, wpe_ref, hin_ref, cs_ref):
    hb = h_ref[...]
    hin = (
        jnp.dot(jnp.cos(hb), wc_ref[...], preferred_element_type=jnp.float32)
        + jnp.dot(jnp.sin(hb), ws_ref[...], preferred_element_type=jnp.float32)
        + bk_ref[...]
        + jnp.dot(pe_ref[...], wpe_ref[...], preferred_element_type=jnp.float32)
    )
    hin_ref[...] = hin

    @pl.when(pl.program_id(0) == 0)
    def _():
        cs_ref[...] = jnp.zeros_like(cs_ref)

    cs_ref[...] += jnp.sum(hin, axis=0, keepdims=True)


def _tc_stage_a(h, pe, wc, ws, bk, wpe):
    n, hd = h.shape
    p = pe.shape[1]
    return pl.pallas_call(
        _stage_a_body,
        grid=(n // RB,),
        in_specs=[
            pl.BlockSpec((RB, hd), lambda i: (i, 0)),
            pl.BlockSpec((RB, p), lambda i: (i, 0)),
            pl.BlockSpec((hd, hd), lambda i: (0, 0)),
            pl.BlockSpec((hd, hd), lambda i: (0, 0)),
            pl.BlockSpec((1, hd), lambda i: (0, 0)),
            pl.BlockSpec((p, hd), lambda i: (0, 0)),
        ],
        out_specs=[
            pl.BlockSpec((RB, hd), lambda i: (i, 0)),
            pl.BlockSpec((1, hd), lambda i: (0, 0)),
        ],
        out_shape=[
            jax.ShapeDtypeStruct((n, hd), jnp.float32),
            jax.ShapeDtypeStruct((1, hd), jnp.float32),
        ],
    )(h, pe, wc, ws, bk.reshape(1, hd), wpe)


def _stage_b_body(inv_n, hin_ref, a0_ref, a1_ref, d0_ref, d1_ref, cs_ref,
                  wl_ref, bl_ref, wg_ref, o_ref):
    deg = jnp.maximum(d0_ref[...] + d1_ref[...], 1.0)
    agg = (a0_ref[...] + a1_ref[...]) / deg
    local = (
        jnp.dot(agg, wl_ref[...], preferred_element_type=jnp.float32)
        + bl_ref[...]
    )
    glob = jnp.dot(cs_ref[...] * inv_n, wg_ref[...],
                   preferred_element_type=jnp.float32)
    o_ref[...] = jnp.maximum(hin_ref[...] + local + glob, 0.0)


def _tc_stage_b(hin, a0, a1, d0, d1, cs, wl, bl, wg):
    n, hd = hin.shape
    return pl.pallas_call(
        functools.partial(_stage_b_body, 1.0 / n),
        grid=(n // RB,),
        in_specs=[
            pl.BlockSpec((RB, hd), lambda i: (i, 0)),
            pl.BlockSpec((RB, hd), lambda i: (i, 0)),
            pl.BlockSpec((RB, hd), lambda i: (i, 0)),
            pl.BlockSpec((RB, 1), lambda i: (i, 0)),
            pl.BlockSpec((RB, 1), lambda i: (i, 0)),
            pl.BlockSpec((1, hd), lambda i: (0, 0)),
            pl.BlockSpec((hd, hd), lambda i: (0, 0)),
            pl.BlockSpec((1, hd), lambda i: (0, 0)),
            pl.BlockSpec((hd, hd), lambda i: (0, 0)),
        ],
        out_specs=pl.BlockSpec((RB, hd), lambda i: (i, 0)),
        out_shape=jax.ShapeDtypeStruct((n, hd), jnp.float32),
    )(hin, a0, a1, d0, d1, cs, wl, bl.reshape(1, hd), wg)


CHUNK = 104


def _make_sc_agg(n, hd, nch, nc, ns):
    zb_step = (n // ns) // 8 * 8
    z_len = (n + 8) - (ns - 1) * zb_step
    o_len = n - (ns - 1) * zb_step
    mesh = plsc.VectorSubcoreMesh(core_axis_name="c", subcore_axis_name="s")

    @functools.partial(
        pl.kernel,
        mesh=mesh,
        out_type=jax.ShapeDtypeStruct((nc, n, hd), jnp.float32),
        scratch_types=[
            pltpu.VMEM((nch * CHUNK,), jnp.int32),
            pltpu.VMEM((nch, CHUNK), jnp.int32),
            pltpu.VMEM((CHUNK, hd), jnp.float32),
            pltpu.VMEM((CHUNK, hd), jnp.float32),
            pltpu.SemaphoreType.DMA,
            pltpu.SemaphoreType.DMA,
            pltpu.SemaphoreType.DMA,
            pltpu.SemaphoreType.DMA,
            pltpu.VMEM_SHARED((n + 8, hd), jnp.float32),
        ],
    )
    def k(h_hbm, src1, dst3, z_hbm, out_hbm, sidx, didx, rows_a, rows_b,
          sga, sgb, ssa, ssb, acc):
        c = lax.axis_index("c")
        s = lax.axis_index("s")
        wid = c * ns + s
        per_tile = nch * CHUNK
        zb = s * zb_step
        pltpu.sync_copy(z_hbm.at[pl.ds(zb, z_len)], acc.at[pl.ds(zb, z_len)])
        pltpu.sync_copy(src1.at[pl.ds(wid * per_tile, per_tile)], sidx)
        pltpu.sync_copy(dst3.at[wid], didx)
        plsc.subcore_barrier()

        def g_start(j, buf, sem):
            pltpu.async_copy(h_hbm.at[sidx.at[pl.ds(j * CHUNK, CHUNK)]],
                             buf, sem)

        def g_wait(buf, sem):
            pltpu.make_async_copy(h_hbm.at[sidx.at[pl.ds(0, CHUNK)]],
                                  buf, sem).wait()

        def s_start(j, buf, sem):
            pltpu.async_copy(buf, acc.at[didx.at[j]], sem, add=True)

        def s_wait(buf, sem):
            pltpu.make_async_copy(buf, acc.at[didx.at[0]], sem).wait()

        g_start(0, rows_a, sga)

        def body(jj, carry):
            j0 = 2 * jj
            j1 = j0 + 1
            g_wait(rows_a, sga)

            @pl.when(jj > 0)
            def _():
                s_wait(rows_b, ssb)

            g_start(j1, rows_b, sgb)
            s_start(j0, rows_a, ssa)
            g_wait(rows_b, sgb)
            s_wait(rows_a, ssa)

            @pl.when(jj < nch // 2 - 1)
            def _():
                g_start(j0 + 2, rows_a, sga)

            s_start(j1, rows_b, ssb)
            return carry

        lax.fori_loop(0, nch // 2, body, 0)
        s_wait(rows_b, ssb)
        plsc.subcore_barrier()
        pltpu.sync_copy(acc.at[pl.ds(zb, o_len)],
                        out_hbm.at[c, pl.ds(zb, o_len)])

    return k


def _make_sc_deg(n, hd, nch, nc, ns):
    zb_step = (n // ns) // 8 * 8
    z_len = (n + 8) - (ns - 1) * zb_step
    o_len = n - (ns - 1) * zb_step
    mesh = plsc.VectorSubcoreMesh(core_axis_name="c", subcore_axis_name="s")

    @functools.partial(
        pl.kernel,
        mesh=mesh,
        out_type=jax.ShapeDtypeStruct((nc, n, hd), jnp.float32),
        scratch_types=[
            pltpu.VMEM((nch, CHUNK), jnp.int32),
            pltpu.VMEM((CHUNK, hd), jnp.float32),
            pltpu.SemaphoreType.DMA,
            pltpu.SemaphoreType.DMA,
            pltpu.VMEM_SHARED((n + 8, hd), jnp.float32),
        ],
    )
    def k(dst3, z_hbm, ones_hbm, out_hbm, didx, ones, ssa, ssb, acc):
        c = lax.axis_index("c")
        s = lax.axis_index("s")
        wid = c * ns + s
        zb = s * zb_step
        pltpu.sync_copy(ones_hbm, ones)
        pltpu.sync_copy(z_hbm.at[pl.ds(zb, z_len)], acc.at[pl.ds(zb, z_len)])
        pltpu.sync_copy(dst3.at[wid], didx)
        plsc.subcore_barrier()

        def s_start(j, sem):
            pltpu.async_copy(ones, acc.at[didx.at[j]], sem, add=True)

        def s_wait(sem):
            pltpu.make_async_copy(ones, acc.at[didx.at[0]], sem).wait()

        s_start(0, ssa)
        s_start(1, ssb)

        def body(jj, carry):
            s_wait(ssa)
            s_start(2 * jj + 2, ssa)
            s_wait(ssb)
            s_start(2 * jj + 3, ssb)
            return carry

        lax.fori_loop(0, nch // 2 - 1, body, 0)
        s_wait(ssa)
        s_wait(ssb)
        plsc.subcore_barrier()
        pltpu.sync_copy(acc.at[pl.ds(zb, o_len)],
                        out_hbm.at[c, pl.ds(zb, o_len)])

    return k


def kernel(x, edge_index, pos_encoding, params):
    n, _ = x.shape
    hd = params['W0'].shape[1]
    e = edge_index.shape[1]
    info = plsc.get_sparse_core_info()
    nc, ns = info.num_cores, info.num_subcores

    nw = nc * ns
    per_tile = -(-e // (nw * 2 * CHUNK)) * 2 * CHUNK
    nch = per_tile // CHUNK
    e_pad = per_tile * nw
    src = edge_index[0].astype(jnp.int32)
    dst = edge_index[1].astype(jnp.int32)
    perm = jnp.argsort(src)
    src_s = src[perm]
    dst_s = dst[perm]
    ppt = e // nw
    pad_t = per_tile - ppt
    src2d = jnp.concatenate(
        [src_s.reshape(nw, ppt), jnp.zeros((nw, pad_t), jnp.int32)], axis=1)
    dst2d = jnp.concatenate(
        [dst_s.reshape(nw, ppt), jnp.full((nw, pad_t), n, jnp.int32)], axis=1)
    src1 = src2d.reshape(-1)
    dst3 = dst2d.reshape(nw, nch, CHUNK)
    zeros2 = jnp.zeros((n + 8, hd), jnp.float32)
    ones_c = jnp.ones((CHUNK, hd), jnp.float32)

    deg_p = _make_sc_deg(n, hd, nch, nc, ns)(dst3, zeros2, ones_c)
    d0 = deg_p[0, :, 0].reshape(n, 1)
    d1 = deg_p[1, :, 0].reshape(n, 1)

    h = _tc_mm_bias(x, params['W0'], params['b0'])
    agg_fn = _make_sc_agg(n, hd, nch, nc, ns)
    for p in params['layers']:
        hin, cs = _tc_stage_a(h, pos_encoding, p['Wc'], p['Ws'], p['bk'],
                              p['Wpe'])
        agg_p = agg_fn(hin, src1, dst3, zeros2)
        h = _tc_stage_b(hin, agg_p[0], agg_p[1], d0, d1, cs, p['Wl'],
                        p['bl'], p['Wg'])
    return _tc_mm_bias(h, params['Wf'], params['bf'])


# speedup vs baseline: 1.5192x; 1.5192x over previous
import functools

import jax
import jax.numpy as jnp
from jax import lax
from jax.experimental import pallas as pl
from jax.experimental.pallas import tpu as pltpu
from jax.experimental.pallas import tpu_sc as plsc

RB = 1000


def _mm_bias_body(x_ref, w_ref, b_ref, o_ref):
    o_ref[...] = (
        jnp.dot(x_ref[...], w_ref[...], preferred_element_type=jnp.float32)
        + b_ref[...]
    )


def _tc_mm_bias(x, w, b):
    n, din = x.shape
    h = w.shape[1]
    return pl.pallas_call(
        _mm_bias_body,
        grid=(n // RB,),
        in_specs=[
            pl.BlockSpec((RB, din), lambda i: (i, 0)),
            pl.BlockSpec((din, h), lambda i: (0, 0)),
            pl.BlockSpec((1, h), lambda i: (0, 0)),
        ],
        out_specs=pl.BlockSpec((RB, h), lambda i: (i, 0)),
        out_shape=jax.ShapeDtypeStruct((n, h), jnp.float32),
    )(x, w, b.reshape(1, h))


def _stage_a_body(h_ref, pe_ref, wc_ref, ws_ref, bk_ref, wpe_ref, hin_ref, cs_ref):
    hb = h_ref[...]
    hin = (
        jnp.dot(jnp.cos(hb), wc_ref[...], preferred_element_type=jnp.float32)
        + jnp.dot(jnp.sin(hb), ws_ref[...], preferred_element_type=jnp.float32)
        + bk_ref[...]
        + jnp.dot(pe_ref[...], wpe_ref[...], preferred_element_type=jnp.float32)
    )
    hin_ref[...] = hin

    @pl.when(pl.program_id(0) == 0)
    def _():
        cs_ref[...] = jnp.zeros_like(cs_ref)

    cs_ref[...] += jnp.sum(hin, axis=0, keepdims=True)


def _tc_stage_a(h, pe, wc, ws, bk, wpe):
    n, hd = h.shape
    p = pe.shape[1]
    return pl.pallas_call(
        _stage_a_body,
        grid=(n // RB,),
        in_specs=[
            pl.BlockSpec((RB, hd), lambda i: (i, 0)),
            pl.BlockSpec((RB, p), lambda i: (i, 0)),
            pl.BlockSpec((hd, hd), lambda i: (0, 0)),
            pl.BlockSpec((hd, hd), lambda i: (0, 0)),
            pl.BlockSpec((1, hd), lambda i: (0, 0)),
            pl.BlockSpec((p, hd), lambda i: (0, 0)),
        ],
        out_specs=[
            pl.BlockSpec((RB, hd), lambda i: (i, 0)),
            pl.BlockSpec((1, hd), lambda i: (0, 0)),
        ],
        out_shape=[
            jax.ShapeDtypeStruct((n, hd), jnp.float32),
            jax.ShapeDtypeStruct((1, hd), jnp.float32),
        ],
    )(h, pe, wc, ws, bk.reshape(1, hd), wpe)


def _stage_b_body(inv_n, hin_ref, a0_ref, a1_ref, d0_ref, d1_ref, cs_ref,
                  wl_ref, bl_ref, wg_ref, o_ref):
    deg = jnp.maximum(d0_ref[...] + d1_ref[...], 1.0)
    agg = (a0_ref[...] + a1_ref[...]) / deg
    local = (
        jnp.dot(agg, wl_ref[...], preferred_element_type=jnp.float32)
        + bl_ref[...]
    )
    glob = jnp.dot(cs_ref[...] * inv_n, wg_ref[...],
                   preferred_element_type=jnp.float32)
    o_ref[...] = jnp.maximum(hin_ref[...] + local + glob, 0.0)


def _tc_stage_b(hin, a0, a1, d0, d1, cs, wl, bl, wg):
    n, hd = hin.shape
    return pl.pallas_call(
        functools.partial(_stage_b_body, 1.0 / n),
        grid=(n // RB,),
        in_specs=[
            pl.BlockSpec((RB, hd), lambda i: (i, 0)),
            pl.BlockSpec((RB, hd), lambda i: (i, 0)),
            pl.BlockSpec((RB, hd), lambda i: (i, 0)),
            pl.BlockSpec((RB, 1), lambda i: (i, 0)),
            pl.BlockSpec((RB, 1), lambda i: (i, 0)),
            pl.BlockSpec((1, hd), lambda i: (0, 0)),
            pl.BlockSpec((hd, hd), lambda i: (0, 0)),
            pl.BlockSpec((1, hd), lambda i: (0, 0)),
            pl.BlockSpec((hd, hd), lambda i: (0, 0)),
        ],
        out_specs=pl.BlockSpec((RB, hd), lambda i: (i, 0)),
        out_shape=jax.ShapeDtypeStruct((n, hd), jnp.float32),
    )(hin, a0, a1, d0, d1, cs, wl, bl.reshape(1, hd), wg)


CHUNK = 104


def _make_sc_agg(n, hd, nch, nc, ns):
    zb_step = (n // ns) // 8 * 8
    z_len = (n + 8) - (ns - 1) * zb_step
    o_len = n - (ns - 1) * zb_step
    mesh = plsc.VectorSubcoreMesh(core_axis_name="c", subcore_axis_name="s")

    @functools.partial(
        pl.kernel,
        mesh=mesh,
        out_type=jax.ShapeDtypeStruct((nc, n, hd), jnp.float32),
        scratch_types=[
            pltpu.VMEM((nch * CHUNK,), jnp.int32),
            pltpu.VMEM((nch, CHUNK), jnp.int32),
            pltpu.VMEM((CHUNK, hd), jnp.float32),
            pltpu.VMEM((CHUNK, hd), jnp.float32),
            pltpu.SemaphoreType.DMA,
            pltpu.SemaphoreType.DMA,
            pltpu.SemaphoreType.DMA,
            pltpu.SemaphoreType.DMA,
            pltpu.VMEM_SHARED((n + 8, hd), jnp.float32),
        ],
    )
    def k(h_hbm, src1, dst3, z_hbm, out_hbm, sidx, didx, rows_a, rows_b,
          sga, sgb, ssa, ssb, acc):
        c = lax.axis_index("c")
        s = lax.axis_index("s")
        wid = c * ns + s
        per_tile = nch * CHUNK
        zb = s * zb_step
        pltpu.sync_copy(z_hbm.at[pl.ds(zb, z_len)], acc.at[pl.ds(zb, z_len)])
        pltpu.sync_copy(src1.at[pl.ds(wid * per_tile, per_tile)], sidx)
        pltpu.sync_copy(dst3.at[wid], didx)
        plsc.subcore_barrier()

        def g_start(j, buf, sem):
            pltpu.async_copy(h_hbm.at[sidx.at[pl.ds(j * CHUNK, CHUNK)]],
                             buf, sem)

        def g_wait(buf, sem):
            pltpu.make_async_copy(h_hbm.at[sidx.at[pl.ds(0, CHUNK)]],
                                  buf, sem).wait()

        def s_start(j, buf, sem):
            pltpu.async_copy(buf, acc.at[didx.at[j]], sem, add=True)

        def s_wait(buf, sem):
            pltpu.make_async_copy(buf, acc.at[didx.at[0]], sem).wait()

        del sgb, ssb, rows_b, s_wait

        def body(j, carry):
            g_start(j, rows_a, sga)
            g_wait(rows_a, sga)
            pltpu.sync_copy(rows_a, acc.at[didx.at[j]], add=True)
            return carry

        lax.fori_loop(0, nch, body, 0)
        del s_start
        plsc.subcore_barrier()
        pltpu.sync_copy(acc.at[pl.ds(zb, o_len)],
                        out_hbm.at[c, pl.ds(zb, o_len)])

    return k


def _make_sc_deg(n, hd, nch, nc, ns):
    zb_step = (n // ns) // 8 * 8
    z_len = (n + 8) - (ns - 1) * zb_step
    o_len = n - (ns - 1) * zb_step
    mesh = plsc.VectorSubcoreMesh(core_axis_name="c", subcore_axis_name="s")

    @functools.partial(
        pl.kernel,
        mesh=mesh,
        out_type=jax.ShapeDtypeStruct((nc, n, hd), jnp.float32),
        scratch_types=[
            pltpu.VMEM((nch, CHUNK), jnp.int32),
            pltpu.VMEM((CHUNK, hd), jnp.float32),
            pltpu.SemaphoreType.DMA,
            pltpu.SemaphoreType.DMA,
            pltpu.VMEM_SHARED((n + 8, hd), jnp.float32),
        ],
    )
    def k(dst3, z_hbm, ones_hbm, out_hbm, didx, ones, ssa, ssb, acc):
        c = lax.axis_index("c")
        s = lax.axis_index("s")
        wid = c * ns + s
        zb = s * zb_step
        pltpu.sync_copy(ones_hbm, ones)
        pltpu.sync_copy(z_hbm.at[pl.ds(zb, z_len)], acc.at[pl.ds(zb, z_len)])
        pltpu.sync_copy(dst3.at[wid], didx)
        plsc.subcore_barrier()

        def s_start(j, sem):
            pltpu.async_copy(ones, acc.at[didx.at[j]], sem, add=True)

        def s_wait(sem):
            pltpu.make_async_copy(ones, acc.at[didx.at[0]], sem).wait()

        s_start(0, ssa)
        s_start(1, ssb)

        def body(jj, carry):
            s_wait(ssa)
            s_start(2 * jj + 2, ssa)
            s_wait(ssb)
            s_start(2 * jj + 3, ssb)
            return carry

        lax.fori_loop(0, nch // 2 - 1, body, 0)
        s_wait(ssa)
        s_wait(ssb)
        plsc.subcore_barrier()
        pltpu.sync_copy(acc.at[pl.ds(zb, o_len)],
                        out_hbm.at[c, pl.ds(zb, o_len)])

    return k


def kernel(x, edge_index, pos_encoding, params):
    n, _ = x.shape
    hd = params['W0'].shape[1]
    e = edge_index.shape[1]
    info = plsc.get_sparse_core_info()
    nc, ns = info.num_cores, info.num_subcores

    nw = nc * ns
    per_tile = -(-e // (nw * 2 * CHUNK)) * 2 * CHUNK
    nch = per_tile // CHUNK
    e_pad = per_tile * nw
    src = edge_index[0].astype(jnp.int32)
    dst = edge_index[1].astype(jnp.int32)
    src_p = jnp.concatenate([src, jnp.zeros((e_pad - e,), jnp.int32)])
    dst_p = jnp.concatenate([dst, jnp.full((e_pad - e,), n, jnp.int32)])
    src1 = src_p.reshape(per_tile, nw).T.reshape(-1)
    dst3 = dst_p.reshape(per_tile, nw).T.reshape(nw, nch, CHUNK)
    zeros2 = jnp.zeros((n + 8, hd), jnp.float32)
    ones_c = jnp.ones((CHUNK, hd), jnp.float32)

    deg_p = _make_sc_deg(n, hd, nch, nc, ns)(dst3, zeros2, ones_c)
    d0 = deg_p[0, :, 0].reshape(n, 1)
    d1 = deg_p[1, :, 0].reshape(n, 1)

    h = _tc_mm_bias(x, params['W0'], params['b0'])
    agg_fn = _make_sc_agg(n, hd, nch, nc, ns)
    for p in params['layers']:
        hin, cs = _tc_stage_a(h, pos_encoding, p['Wc'], p['Ws'], p['bk'],
                              p['Wpe'])
        agg_p = agg_fn(hin, src1, dst3, zeros2)
        h = _tc_stage_b(hin, agg_p[0], agg_p[1], d0, d1, cs, p['Wl'],
                        p['bl'], p['Wg'])
    return _tc_mm_bias(h, params['Wf'], params['bf'])


# speedup vs baseline: 1.9297x; 1.2702x over previous
import functools

import jax
import jax.numpy as jnp
from jax import lax
from jax.experimental import pallas as pl
from jax.experimental.pallas import tpu as pltpu
from jax.experimental.pallas import tpu_sc as plsc

RB = 1000


def _mm_bias_body(x_ref, w_ref, b_ref, o_ref):
    o_ref[...] = (
        jnp.dot(x_ref[...], w_ref[...], preferred_element_type=jnp.float32)
        + b_ref[...]
    )


def _tc_mm_bias(x, w, b):
    n, din = x.shape
    h = w.shape[1]
    return pl.pallas_call(
        _mm_bias_body,
        grid=(n // RB,),
        in_specs=[
            pl.BlockSpec((RB, din), lambda i: (i, 0)),
            pl.BlockSpec((din, h), lambda i: (0, 0)),
            pl.BlockSpec((1, h), lambda i: (0, 0)),
        ],
        out_specs=pl.BlockSpec((RB, h), lambda i: (i, 0)),
        out_shape=jax.ShapeDtypeStruct((n, h), jnp.float32),
    )(x, w, b.reshape(1, h))


def _stage_a_body(h_ref, pe_ref, wc_ref, ws_ref, bk_ref, wpe_ref, hin_ref, cs_ref):
    hb = h_ref[...]
    hin = (
        jnp.dot(jnp.cos(hb), wc_ref[...], preferred_element_type=jnp.float32)
        + jnp.dot(jnp.sin(hb), ws_ref[...], preferred_element_type=jnp.float32)
        + bk_ref[...]
        + jnp.dot(pe_ref[...], wpe_ref[...], preferred_element_type=jnp.float32)
    )
    hin_ref[...] = hin

    @pl.when(pl.program_id(0) == 0)
    def _():
        cs_ref[...] = jnp.zeros_like(cs_ref)

    cs_ref[...] += jnp.sum(hin, axis=0, keepdims=True)


def _tc_stage_a(h, pe, wc, ws, bk, wpe):
    n, hd = h.shape
    p = pe.shape[1]
    return pl.pallas_call(
        _stage_a_body,
        grid=(n // RB,),
        in_specs=[
            pl.BlockSpec((RB, hd), lambda i: (i, 0)),
            pl.BlockSpec((RB, p), lambda i: (i, 0)),
            pl.BlockSpec((hd, hd), lambda i: (0, 0)),
            pl.BlockSpec((hd, hd), lambda i: (0, 0)),
            pl.BlockSpec((1, hd), lambda i: (0, 0)),
            pl.BlockSpec((p, hd), lambda i: (0, 0)),
        ],
        out_specs=[
            pl.BlockSpec((RB, hd), lambda i: (i, 0)),
            pl.BlockSpec((1, hd), lambda i: (0, 0)),
        ],
        out_shape=[
            jax.ShapeDtypeStruct((n, hd), jnp.float32),
            jax.ShapeDtypeStruct((1, hd), jnp.float32),
        ],
    )(h, pe, wc, ws, bk.reshape(1, hd), wpe)


def _stage_b_body(inv_n, hin_ref, a0_ref, a1_ref, d0_ref, d1_ref, cs_ref,
                  wl_ref, bl_ref, wg_ref, o_ref):
    deg = jnp.maximum(d0_ref[...] + d1_ref[...], 1.0)
    agg = (a0_ref[...] + a1_ref[...]) / deg
    local = (
        jnp.dot(agg, wl_ref[...], preferred_element_type=jnp.float32)
        + bl_ref[...]
    )
    glob = jnp.dot(cs_ref[...] * inv_n, wg_ref[...],
                   preferred_element_type=jnp.float32)
    o_ref[...] = jnp.maximum(hin_ref[...] + local + glob, 0.0)


def _tc_stage_b(hin, a0, a1, d0, d1, cs, wl, bl, wg):
    n, hd = hin.shape
    return pl.pallas_call(
        functools.partial(_stage_b_body, 1.0 / n),
        grid=(n // RB,),
        in_specs=[
            pl.BlockSpec((RB, hd), lambda i: (i, 0)),
            pl.BlockSpec((RB, hd), lambda i: (i, 0)),
            pl.BlockSpec((RB, hd), lambda i: (i, 0)),
            pl.BlockSpec((RB, 1), lambda i: (i, 0)),
            pl.BlockSpec((RB, 1), lambda i: (i, 0)),
            pl.BlockSpec((1, hd), lambda i: (0, 0)),
            pl.BlockSpec((hd, hd), lambda i: (0, 0)),
            pl.BlockSpec((1, hd), lambda i: (0, 0)),
            pl.BlockSpec((hd, hd), lambda i: (0, 0)),
        ],
        out_specs=pl.BlockSpec((RB, hd), lambda i: (i, 0)),
        out_shape=jax.ShapeDtypeStruct((n, hd), jnp.float32),
    )(hin, a0, a1, d0, d1, cs, wl, bl.reshape(1, hd), wg)


def _pick_chunk(per_tile):
    for c in range(128, 7, -8):
        if per_tile % c == 0:
            return c
    return None


def _make_sc_agg(n, hd, nch, chunk, nc, ns):
    zb_step = (n // ns) // 8 * 8
    z_len = (n + 8) - (ns - 1) * zb_step
    o_len = n - (ns - 1) * zb_step
    mesh = plsc.VectorSubcoreMesh(core_axis_name="c", subcore_axis_name="s")

    @functools.partial(
        pl.kernel,
        mesh=mesh,
        out_type=jax.ShapeDtypeStruct((nc, n, hd), jnp.float32),
        scratch_types=[
            pltpu.VMEM((chunk,), jnp.int32),
            pltpu.VMEM((chunk,), jnp.int32),
            pltpu.VMEM((chunk, hd), jnp.float32),
            pltpu.SemaphoreType.DMA,
            pltpu.VMEM_SHARED((n + 8, hd), jnp.float32),
        ],
    )
    def k(h_hbm, src1, dst1, z_hbm, out_hbm, sidx, didx, rows, sem, acc):
        c = lax.axis_index("c")
        s = lax.axis_index("s")
        wid = c * ns + s
        per_tile = nch * chunk
        zb = s * zb_step
        pltpu.sync_copy(z_hbm.at[pl.ds(zb, z_len)], acc.at[pl.ds(zb, z_len)])
        plsc.subcore_barrier()

        def body(j, carry):
            base = wid * per_tile + j * chunk
            pltpu.sync_copy(src1.at[pl.ds(base, chunk)], sidx)
            pltpu.sync_copy(dst1.at[pl.ds(base, chunk)], didx)
            pltpu.async_copy(h_hbm.at[sidx], rows, sem).wait()
            pltpu.sync_copy(rows, acc.at[didx], add=True)
            return carry

        lax.fori_loop(0, nch, body, 0)
        plsc.subcore_barrier()
        pltpu.sync_copy(acc.at[pl.ds(zb, o_len)],
                        out_hbm.at[c, pl.ds(zb, o_len)])

    return k


def _make_sc_deg(n, hd, nch, chunk, nc, ns):
    zb_step = (n // ns) // 8 * 8
    z_len = (n + 8) - (ns - 1) * zb_step
    o_len = n - (ns - 1) * zb_step
    mesh = plsc.VectorSubcoreMesh(core_axis_name="c", subcore_axis_name="s")

    @functools.partial(
        pl.kernel,
        mesh=mesh,
        out_type=jax.ShapeDtypeStruct((nc, n, hd), jnp.float32),
        scratch_types=[
            pltpu.VMEM((nch, chunk), jnp.int32),
            pltpu.VMEM((chunk, hd), jnp.float32),
            pltpu.SemaphoreType.DMA,
            pltpu.SemaphoreType.DMA,
            pltpu.VMEM_SHARED((n + 8, hd), jnp.float32),
        ],
    )
    def k(dst3, z_hbm, ones_hbm, out_hbm, didx, ones, ssa, ssb, acc):
        c = lax.axis_index("c")
        s = lax.axis_index("s")
        wid = c * ns + s
        zb = s * zb_step
        pltpu.sync_copy(ones_hbm, ones)
        pltpu.sync_copy(z_hbm.at[pl.ds(zb, z_len)], acc.at[pl.ds(zb, z_len)])
        pltpu.sync_copy(dst3.at[wid], didx)
        plsc.subcore_barrier()

        def s_start(j, sem):
            pltpu.async_copy(ones, acc.at[didx.at[j]], sem, add=True)

        def s_wait(sem):
            pltpu.make_async_copy(ones, acc.at[didx.at[0]], sem).wait()

        s_start(0, ssa)
        s_start(1, ssb)

        def body(jj, carry):
            s_wait(ssa)
            s_start(2 * jj + 2, ssa)
            s_wait(ssb)
            s_start(2 * jj + 3, ssb)
            return carry

        lax.fori_loop(0, (nch - 2) // 2, body, 0)
        if (nch - 2) % 2:
            s_wait(ssa)
            s_start(nch - 1, ssa)
        s_wait(ssa)
        s_wait(ssb)
        plsc.subcore_barrier()
        pltpu.sync_copy(acc.at[pl.ds(zb, o_len)],
                        out_hbm.at[c, pl.ds(zb, o_len)])

    return k


def kernel(x, edge_index, pos_encoding, params):
    n, _ = x.shape
    hd = params['W0'].shape[1]
    e = edge_index.shape[1]
    info = plsc.get_sparse_core_info()
    nc, ns = info.num_cores, info.num_subcores

    nw = nc * ns
    src = edge_index[0].astype(jnp.int32)
    dst = edge_index[1].astype(jnp.int32)
    chunk = _pick_chunk(e // nw) if e % nw == 0 else None
    if chunk is not None:
        per_tile = e // nw
        nch = per_tile // chunk
        src1, dst1 = src, dst
    else:
        chunk = 128
        per_tile = -(-e // (nw * chunk)) * chunk
        nch = per_tile // chunk
        e_pad = per_tile * nw
        src1 = jnp.concatenate(
            [src, jnp.zeros((e_pad - e,), jnp.int32)]
        ).reshape(per_tile, nw).T.reshape(-1)
        dst1 = jnp.concatenate(
            [dst, jnp.full((e_pad - e,), n, jnp.int32)]
        ).reshape(per_tile, nw).T.reshape(-1)
    dst3 = dst1.reshape(nw, nch, chunk)
    zeros2 = jnp.zeros((n + 8, hd), jnp.float32)
    ones_c = jnp.ones((chunk, hd), jnp.float32)

    deg_p = _make_sc_deg(n, hd, nch, chunk, nc, ns)(dst3, zeros2, ones_c)
    d0 = deg_p[0, :, 0].reshape(n, 1)
    d1 = deg_p[1, :, 0].reshape(n, 1)

    h = _tc_mm_bias(x, params['W0'], params['b0'])
    agg_fn = _make_sc_agg(n, hd, nch, chunk, nc, ns)
    for p in params['layers']:
        hin, cs = _tc_stage_a(h, pos_encoding, p['Wc'], p['Ws'], p['bk'],
                              p['Wpe'])
        agg_p = agg_fn(hin, src1, dst1, zeros2)
        h = _tc_stage_b(hin, agg_p[0], agg_p[1], d0, d1, cs, p['Wl'],
                        p['bl'], p['Wg'])
    return _tc_mm_bias(h, params['Wf'], params['bf'])


# speedup vs baseline: 1.9733x; 1.0226x over previous
import functools

import jax
import jax.numpy as jnp
from jax import lax
from jax.experimental import pallas as pl
from jax.experimental.pallas import tpu as pltpu
from jax.experimental.pallas import tpu_sc as plsc

RB = 1000


def _kan_pe(h, pe_ref, wc_ref, ws_ref, bk_ref, wpe_ref):
    return (
        jnp.dot(jnp.cos(h), wc_ref[...], preferred_element_type=jnp.float32)
        + jnp.dot(jnp.sin(h), ws_ref[...], preferred_element_type=jnp.float32)
        + bk_ref[...]
        + jnp.dot(pe_ref[...], wpe_ref[...], preferred_element_type=jnp.float32)
    )


def _combine(inv_n, hin_ref, a0_ref, a1_ref, d0_ref, d1_ref, cs_ref,
             wl_ref, bl_ref, wg_ref):
    deg = jnp.maximum(d0_ref[...] + d1_ref[...], 1.0)
    agg = (a0_ref[...] + a1_ref[...]) / deg
    local = (
        jnp.dot(agg, wl_ref[...], preferred_element_type=jnp.float32)
        + bl_ref[...]
    )
    glob = jnp.dot(cs_ref[...] * inv_n, wg_ref[...],
                   preferred_element_type=jnp.float32)
    return jnp.maximum(hin_ref[...] + local + glob, 0.0)


def _accum_colsum(hin, cs_ref):
    @pl.when(pl.program_id(0) == 0)
    def _():
        cs_ref[...] = jnp.zeros_like(cs_ref)

    cs_ref[...] += jnp.sum(hin, axis=0, keepdims=True)


def _stage_a0_body(x_ref, pe_ref, w0_ref, b0_ref, wc_ref, ws_ref, bk_ref,
                   wpe_ref, hin_ref, cs_ref):
    h = (jnp.dot(x_ref[...], w0_ref[...], preferred_element_type=jnp.float32)
         + b0_ref[...])
    hin = _kan_pe(h, pe_ref, wc_ref, ws_ref, bk_ref, wpe_ref)
    hin_ref[...] = hin
    _accum_colsum(hin, cs_ref)


def _tc_stage_a0(x, pe, w0, b0, wc, ws, bk, wpe):
    n, din = x.shape
    hd = w0.shape[1]
    p = pe.shape[1]
    full = lambda a, b: pl.BlockSpec((a, b), lambda i: (0, 0))
    return pl.pallas_call(
        _stage_a0_body,
        grid=(n // RB,),
        in_specs=[
            pl.BlockSpec((RB, din), lambda i: (i, 0)),
            pl.BlockSpec((RB, p), lambda i: (i, 0)),
            full(din, hd), full(1, hd), full(hd, hd), full(hd, hd),
            full(1, hd), full(p, hd),
        ],
        out_specs=[
            pl.BlockSpec((RB, hd), lambda i: (i, 0)),
            pl.BlockSpec((1, hd), lambda i: (0, 0)),
        ],
        out_shape=[
            jax.ShapeDtypeStruct((n, hd), jnp.float32),
            jax.ShapeDtypeStruct((1, hd), jnp.float32),
        ],
    )(x, pe, w0, b0.reshape(1, hd), wc, ws, bk.reshape(1, hd), wpe)


def _stage_ba_body(inv_n, hin_ref, a0_ref, a1_ref, d0_ref, d1_ref, cs_ref,
                   wl_ref, bl_ref, wg_ref, pe_ref, wc_ref, ws_ref, bk_ref,
                   wpe_ref, hin2_ref, cs2_ref):
    h = _combine(inv_n, hin_ref, a0_ref, a1_ref, d0_ref, d1_ref, cs_ref,
                 wl_ref, bl_ref, wg_ref)
    hin2 = _kan_pe(h, pe_ref, wc_ref, ws_ref, bk_ref, wpe_ref)
    hin2_ref[...] = hin2
    _accum_colsum(hin2, cs2_ref)


def _tc_stage_ba(hin, a0, a1, d0, d1, cs, wl, bl, wg, pe, wc, ws, bk, wpe):
    n, hd = hin.shape
    p = pe.shape[1]
    row = lambda b: pl.BlockSpec((RB, b), lambda i: (i, 0))
    full = lambda a, b: pl.BlockSpec((a, b), lambda i: (0, 0))
    return pl.pallas_call(
        functools.partial(_stage_ba_body, 1.0 / n),
        grid=(n // RB,),
        in_specs=[
            row(hd), row(hd), row(hd), row(1), row(1), full(1, hd),
            full(hd, hd), full(1, hd), full(hd, hd),
            row(p), full(hd, hd), full(hd, hd), full(1, hd), full(p, hd),
        ],
        out_specs=[
            pl.BlockSpec((RB, hd), lambda i: (i, 0)),
            pl.BlockSpec((1, hd), lambda i: (0, 0)),
        ],
        out_shape=[
            jax.ShapeDtypeStruct((n, hd), jnp.float32),
            jax.ShapeDtypeStruct((1, hd), jnp.float32),
        ],
    )(hin, a0, a1, d0, d1, cs, wl, bl.reshape(1, hd), wg,
      pe, wc, ws, bk.reshape(1, hd), wpe)


def _stage_bf_body(inv_n, hin_ref, a0_ref, a1_ref, d0_ref, d1_ref, cs_ref,
                   wl_ref, bl_ref, wg_ref, wf_ref, bf_ref, o_ref):
    h = _combine(inv_n, hin_ref, a0_ref, a1_ref, d0_ref, d1_ref, cs_ref,
                 wl_ref, bl_ref, wg_ref)
    o_ref[...] = (
        jnp.dot(h, wf_ref[...], preferred_element_type=jnp.float32)
        + bf_ref[...]
    )


def _tc_stage_bf(hin, a0, a1, d0, d1, cs, wl, bl, wg, wf, bf):
    n, hd = hin.shape
    dout = wf.shape[1]
    row = lambda b: pl.BlockSpec((RB, b), lambda i: (i, 0))
    full = lambda a, b: pl.BlockSpec((a, b), lambda i: (0, 0))
    return pl.pallas_call(
        functools.partial(_stage_bf_body, 1.0 / n),
        grid=(n // RB,),
        in_specs=[
            row(hd), row(hd), row(hd), row(1), row(1), full(1, hd),
            full(hd, hd), full(1, hd), full(hd, hd),
            full(hd, dout), full(1, dout),
        ],
        out_specs=pl.BlockSpec((RB, dout), lambda i: (i, 0)),
        out_shape=jax.ShapeDtypeStruct((n, dout), jnp.float32),
    )(hin, a0, a1, d0, d1, cs, wl, bl.reshape(1, hd), wg,
      wf, bf.reshape(1, dout))


def _pick_chunk(per_tile):
    for c in range(128, 7, -8):
        if per_tile % c == 0:
            return c
    return None


def _make_sc_agg(n, hd, nch, chunk, nc, ns):
    zb_step = (n // ns) // 8 * 8
    z_len = (n + 8) - (ns - 1) * zb_step
    o_len = n - (ns - 1) * zb_step
    mesh = plsc.VectorSubcoreMesh(core_axis_name="c", subcore_axis_name="s")

    @functools.partial(
        pl.kernel,
        mesh=mesh,
        out_type=jax.ShapeDtypeStruct((nc, n, hd), jnp.float32),
        scratch_types=[
            pltpu.VMEM((chunk,), jnp.int32),
            pltpu.VMEM((chunk,), jnp.int32),
            pltpu.VMEM((chunk, hd), jnp.float32),
            pltpu.SemaphoreType.DMA,
            pltpu.VMEM_SHARED((n + 8, hd), jnp.float32),
        ],
    )
    def k(h_hbm, src1, dst1, z_hbm, out_hbm, sidx, didx, rows, sem, acc):
        c = lax.axis_index("c")
        s = lax.axis_index("s")
        wid = c * ns + s
        per_tile = nch * chunk
        zb = s * zb_step
        pltpu.sync_copy(z_hbm.at[pl.ds(zb, z_len)], acc.at[pl.ds(zb, z_len)])
        plsc.subcore_barrier()

        def body(j, carry):
            base = wid * per_tile + j * chunk
            pltpu.sync_copy(src1.at[pl.ds(base, chunk)], sidx)
            pltpu.sync_copy(dst1.at[pl.ds(base, chunk)], didx)
            pltpu.async_copy(h_hbm.at[sidx], rows, sem).wait()
            pltpu.sync_copy(rows, acc.at[didx], add=True)
            return carry

        lax.fori_loop(0, nch, body, 0)
        plsc.subcore_barrier()
        pltpu.sync_copy(acc.at[pl.ds(zb, o_len)],
                        out_hbm.at[c, pl.ds(zb, o_len)])

    return k


def _make_sc_deg(n, hd, nch, chunk, nc, ns):
    zb_step = (n // ns) // 8 * 8
    z_len = (n + 8) - (ns - 1) * zb_step
    o_len = n - (ns - 1) * zb_step
    mesh = plsc.VectorSubcoreMesh(core_axis_name="c", subcore_axis_name="s")

    @functools.partial(
        pl.kernel,
        mesh=mesh,
        out_type=jax.ShapeDtypeStruct((nc, n, hd), jnp.float32),
        scratch_types=[
            pltpu.VMEM((nch, chunk), jnp.int32),
            pltpu.VMEM((chunk, hd), jnp.float32),
            pltpu.SemaphoreType.DMA,
            pltpu.SemaphoreType.DMA,
            pltpu.VMEM_SHARED((n + 8, hd), jnp.float32),
        ],
    )
    def k(dst3, z_hbm, ones_hbm, out_hbm, didx, ones, ssa, ssb, acc):
        c = lax.axis_index("c")
        s = lax.axis_index("s")
        wid = c * ns + s
        zb = s * zb_step
        pltpu.sync_copy(ones_hbm, ones)
        pltpu.sync_copy(z_hbm.at[pl.ds(zb, z_len)], acc.at[pl.ds(zb, z_len)])
        pltpu.sync_copy(dst3.at[wid], didx)
        plsc.subcore_barrier()

        def s_start(j, sem):
            pltpu.async_copy(ones, acc.at[didx.at[j]], sem, add=True)

        def s_wait(sem):
            pltpu.make_async_copy(ones, acc.at[didx.at[0]], sem).wait()

        s_start(0, ssa)
        s_start(1, ssb)

        def body(jj, carry):
            s_wait(ssa)
            s_start(2 * jj + 2, ssa)
            s_wait(ssb)
            s_start(2 * jj + 3, ssb)
            return carry

        lax.fori_loop(0, (nch - 2) // 2, body, 0)
        if (nch - 2) % 2:
            s_wait(ssa)
            s_start(nch - 1, ssa)
        s_wait(ssa)
        s_wait(ssb)
        plsc.subcore_barrier()
        pltpu.sync_copy(acc.at[pl.ds(zb, o_len)],
                        out_hbm.at[c, pl.ds(zb, o_len)])

    return k


def kernel(x, edge_index, pos_encoding, params):
    n, _ = x.shape
    hd = params['W0'].shape[1]
    e = edge_index.shape[1]
    info = plsc.get_sparse_core_info()
    nc, ns = info.num_cores, info.num_subcores

    nw = nc * ns
    src = edge_index[0].astype(jnp.int32)
    dst = edge_index[1].astype(jnp.int32)
    chunk = _pick_chunk(e // nw) if e % nw == 0 else None
    if chunk is not None:
        per_tile = e // nw
        nch = per_tile // chunk
        src1, dst1 = src, dst
    else:
        chunk = 128
        per_tile = -(-e // (nw * chunk)) * chunk
        nch = per_tile // chunk
        e_pad = per_tile * nw
        src1 = jnp.concatenate(
            [src, jnp.zeros((e_pad - e,), jnp.int32)]
        ).reshape(per_tile, nw).T.reshape(-1)
        dst1 = jnp.concatenate(
            [dst, jnp.full((e_pad - e,), n, jnp.int32)]
        ).reshape(per_tile, nw).T.reshape(-1)
    dst3 = dst1.reshape(nw, nch, chunk)
    zeros2 = jnp.zeros((n + 8, hd), jnp.float32)
    ones_c = jnp.ones((chunk, hd), jnp.float32)

    deg_p = _make_sc_deg(n, hd, nch, chunk, nc, ns)(dst3, zeros2, ones_c)
    d0 = deg_p[0, :, 0].reshape(n, 1)
    d1 = deg_p[1, :, 0].reshape(n, 1)

    agg_fn = _make_sc_agg(n, hd, nch, chunk, nc, ns)
    lp = params['layers']
    hin, cs = _tc_stage_a0(x, pos_encoding, params['W0'], params['b0'],
                           lp[0]['Wc'], lp[0]['Ws'], lp[0]['bk'],
                           lp[0]['Wpe'])
    for i in range(len(lp) - 1):
        p, q = lp[i], lp[i + 1]
        agg_p = agg_fn(hin, src1, dst1, zeros2)
        hin, cs = _tc_stage_ba(hin, agg_p[0], agg_p[1], d0, d1, cs,
                               p['Wl'], p['bl'], p['Wg'], pos_encoding,
                               q['Wc'], q['Ws'], q['bk'], q['Wpe'])
    p = lp[-1]
    agg_p = agg_fn(hin, src1, dst1, zeros2)
    return _tc_stage_bf(hin, agg_p[0], agg_p[1], d0, d1, cs, p['Wl'],
                        p['bl'], p['Wg'], params['Wf'], params['bf'])


# speedup vs baseline: 2.2770x; 1.1539x over previous
import functools

import jax
import jax.numpy as jnp
from jax import lax
from jax.experimental import pallas as pl
from jax.experimental.pallas import tpu as pltpu
from jax.experimental.pallas import tpu_sc as plsc

RB = 1000


def _kan_pe(h, pe_ref, wc_ref, ws_ref, bk_ref, wpe_ref):
    return (
        jnp.dot(jnp.cos(h), wc_ref[...], preferred_element_type=jnp.float32)
        + jnp.dot(jnp.sin(h), ws_ref[...], preferred_element_type=jnp.float32)
        + bk_ref[...]
        + jnp.dot(pe_ref[...], wpe_ref[...], preferred_element_type=jnp.float32)
    )


def _combine(inv_n, hin_ref, a0_ref, a1_ref, d0_ref, d1_ref, cs_ref,
             wl_ref, bl_ref, wg_ref):
    deg = jnp.maximum(d0_ref[...] + d1_ref[...], 1.0)
    agg = (a0_ref[...] + a1_ref[...]) / deg
    local = (
        jnp.dot(agg, wl_ref[...], preferred_element_type=jnp.float32)
        + bl_ref[...]
    )
    glob = jnp.dot(cs_ref[...] * inv_n, wg_ref[...],
                   preferred_element_type=jnp.float32)
    return jnp.maximum(hin_ref[...] + local + glob, 0.0)


def _accum_colsum(hin, cs_ref):
    @pl.when(pl.program_id(0) == 0)
    def _():
        cs_ref[...] = jnp.zeros_like(cs_ref)

    cs_ref[...] += jnp.sum(hin, axis=0, keepdims=True)


def _stage_a0_body(x_ref, pe_ref, w0_ref, b0_ref, wc_ref, ws_ref, bk_ref,
                   wpe_ref, hin_ref, cs_ref):
    h = (jnp.dot(x_ref[...], w0_ref[...], preferred_element_type=jnp.float32)
         + b0_ref[...])
    hin = _kan_pe(h, pe_ref, wc_ref, ws_ref, bk_ref, wpe_ref)
    hin_ref[...] = hin
    _accum_colsum(hin, cs_ref)


def _tc_stage_a0(x, pe, w0, b0, wc, ws, bk, wpe):
    n, din = x.shape
    hd = w0.shape[1]
    p = pe.shape[1]
    full = lambda a, b: pl.BlockSpec((a, b), lambda i: (0, 0))
    return pl.pallas_call(
        _stage_a0_body,
        grid=(n // RB,),
        in_specs=[
            pl.BlockSpec((RB, din), lambda i: (i, 0)),
            pl.BlockSpec((RB, p), lambda i: (i, 0)),
            full(din, hd), full(1, hd), full(hd, hd), full(hd, hd),
            full(1, hd), full(p, hd),
        ],
        out_specs=[
            pl.BlockSpec((RB, hd), lambda i: (i, 0)),
            pl.BlockSpec((1, hd), lambda i: (0, 0)),
        ],
        out_shape=[
            jax.ShapeDtypeStruct((n, hd), jnp.float32),
            jax.ShapeDtypeStruct((1, hd), jnp.float32),
        ],
    )(x, pe, w0, b0.reshape(1, hd), wc, ws, bk.reshape(1, hd), wpe)


def _stage_ba_body(inv_n, hin_ref, a0_ref, a1_ref, d0_ref, d1_ref, cs_ref,
                   wl_ref, bl_ref, wg_ref, pe_ref, wc_ref, ws_ref, bk_ref,
                   wpe_ref, hin2_ref, cs2_ref):
    h = _combine(inv_n, hin_ref, a0_ref, a1_ref, d0_ref, d1_ref, cs_ref,
                 wl_ref, bl_ref, wg_ref)
    hin2 = _kan_pe(h, pe_ref, wc_ref, ws_ref, bk_ref, wpe_ref)
    hin2_ref[...] = hin2
    _accum_colsum(hin2, cs2_ref)


def _tc_stage_ba(hin, a0, a1, d0, d1, cs, wl, bl, wg, pe, wc, ws, bk, wpe):
    n, hd = hin.shape
    p = pe.shape[1]
    row = lambda b: pl.BlockSpec((RB, b), lambda i: (i, 0))
    full = lambda a, b: pl.BlockSpec((a, b), lambda i: (0, 0))
    return pl.pallas_call(
        functools.partial(_stage_ba_body, 1.0 / n),
        grid=(n // RB,),
        in_specs=[
            row(hd), row(hd), row(hd), row(1), row(1), full(1, hd),
            full(hd, hd), full(1, hd), full(hd, hd),
            row(p), full(hd, hd), full(hd, hd), full(1, hd), full(p, hd),
        ],
        out_specs=[
            pl.BlockSpec((RB, hd), lambda i: (i, 0)),
            pl.BlockSpec((1, hd), lambda i: (0, 0)),
        ],
        out_shape=[
            jax.ShapeDtypeStruct((n, hd), jnp.float32),
            jax.ShapeDtypeStruct((1, hd), jnp.float32),
        ],
    )(hin, a0, a1, d0, d1, cs, wl, bl.reshape(1, hd), wg,
      pe, wc, ws, bk.reshape(1, hd), wpe)


def _stage_bf_body(inv_n, hin_ref, a0_ref, a1_ref, d0_ref, d1_ref, cs_ref,
                   wl_ref, bl_ref, wg_ref, wf_ref, bf_ref, o_ref):
    h = _combine(inv_n, hin_ref, a0_ref, a1_ref, d0_ref, d1_ref, cs_ref,
                 wl_ref, bl_ref, wg_ref)
    o_ref[...] = (
        jnp.dot(h, wf_ref[...], preferred_element_type=jnp.float32)
        + bf_ref[...]
    )


def _tc_stage_bf(hin, a0, a1, d0, d1, cs, wl, bl, wg, wf, bf):
    n, hd = hin.shape
    dout = wf.shape[1]
    row = lambda b: pl.BlockSpec((RB, b), lambda i: (i, 0))
    full = lambda a, b: pl.BlockSpec((a, b), lambda i: (0, 0))
    return pl.pallas_call(
        functools.partial(_stage_bf_body, 1.0 / n),
        grid=(n // RB,),
        in_specs=[
            row(hd), row(hd), row(hd), row(1), row(1), full(1, hd),
            full(hd, hd), full(1, hd), full(hd, hd),
            full(hd, dout), full(1, dout),
        ],
        out_specs=pl.BlockSpec((RB, dout), lambda i: (i, 0)),
        out_shape=jax.ShapeDtypeStruct((n, dout), jnp.float32),
    )(hin, a0, a1, d0, d1, cs, wl, bl.reshape(1, hd), wg,
      wf, bf.reshape(1, dout))


def _pick_chunk(per_tile):
    for c in range(128, 7, -8):
        if per_tile % c == 0:
            return c
    return None


def _make_sc_agg(n, hd, nch, chunk, nc, ns):
    zb_step = (n // ns) // 8 * 8
    z_len = (n + 8) - (ns - 1) * zb_step
    o_len = n - (ns - 1) * zb_step
    mesh = plsc.VectorSubcoreMesh(core_axis_name="c", subcore_axis_name="s")

    @functools.partial(
        pl.kernel,
        mesh=mesh,
        out_type=jax.ShapeDtypeStruct((nc, n, hd), jnp.float32),
        scratch_types=[
            pltpu.VMEM((chunk,), jnp.int32),
            pltpu.VMEM((chunk,), jnp.int32),
            pltpu.VMEM((chunk,), jnp.int32),
            pltpu.VMEM((chunk,), jnp.int32),
            pltpu.VMEM((chunk, hd), jnp.float32),
            pltpu.VMEM((chunk, hd), jnp.float32),
            pltpu.SemaphoreType.DMA,
            pltpu.SemaphoreType.DMA,
            pltpu.SemaphoreType.DMA,
            pltpu.VMEM_SHARED((n + 8, hd), jnp.float32),
        ],
    )
    def k(h_hbm, src1, dst1, z_hbm, out_hbm, sidx_a, didx_a, sidx_b, didx_b,
          rows_a, rows_b, sg, ssa, ssb, acc):
        c = lax.axis_index("c")
        s = lax.axis_index("s")
        wid = c * ns + s
        per_tile = nch * chunk
        zb = s * zb_step
        pltpu.sync_copy(z_hbm.at[pl.ds(zb, z_len)], acc.at[pl.ds(zb, z_len)])
        plsc.subcore_barrier()

        def do_chunk(j, sidx, didx, rows, ssem, first):
            base = wid * per_tile + j * chunk
            if not first:
                pltpu.make_async_copy(rows, acc.at[didx.at[pl.ds(0, chunk)]],
                                      ssem).wait()
            pltpu.sync_copy(src1.at[pl.ds(base, chunk)], sidx)
            pltpu.sync_copy(dst1.at[pl.ds(base, chunk)], didx)
            pltpu.async_copy(h_hbm.at[sidx], rows, sg).wait()
            pltpu.async_copy(rows, acc.at[didx], ssem, add=True)

        do_chunk(0, sidx_a, didx_a, rows_a, ssa, True)
        do_chunk(1, sidx_b, didx_b, rows_b, ssb, True)

        def body(jj, carry):
            do_chunk(2 * jj, sidx_a, didx_a, rows_a, ssa, False)
            do_chunk(2 * jj + 1, sidx_b, didx_b, rows_b, ssb, False)
            return carry

        lax.fori_loop(1, nch // 2, body, 0)
        if nch % 2:
            do_chunk(nch - 1, sidx_a, didx_a, rows_a, ssa, False)
        pltpu.make_async_copy(rows_a, acc.at[didx_a.at[pl.ds(0, chunk)]],
                              ssa).wait()
        pltpu.make_async_copy(rows_b, acc.at[didx_b.at[pl.ds(0, chunk)]],
                              ssb).wait()
        plsc.subcore_barrier()
        pltpu.sync_copy(acc.at[pl.ds(zb, o_len)],
                        out_hbm.at[c, pl.ds(zb, o_len)])

    return k


def _make_sc_deg(n, hd, nch, chunk, nc, ns):
    zb_step = (n // ns) // 8 * 8
    z_len = (n + 8) - (ns - 1) * zb_step
    o_len = n - (ns - 1) * zb_step
    mesh = plsc.VectorSubcoreMesh(core_axis_name="c", subcore_axis_name="s")

    @functools.partial(
        pl.kernel,
        mesh=mesh,
        out_type=jax.ShapeDtypeStruct((nc, n, hd), jnp.float32),
        scratch_types=[
            pltpu.VMEM((nch, chunk), jnp.int32),
            pltpu.VMEM((chunk, hd), jnp.float32),
            pltpu.SemaphoreType.DMA,
            pltpu.SemaphoreType.DMA,
            pltpu.VMEM_SHARED((n + 8, hd), jnp.float32),
        ],
    )
    def k(dst3, z_hbm, ones_hbm, out_hbm, didx, ones, ssa, ssb, acc):
        c = lax.axis_index("c")
        s = lax.axis_index("s")
        wid = c * ns + s
        zb = s * zb_step
        pltpu.sync_copy(ones_hbm, ones)
        pltpu.sync_copy(z_hbm.at[pl.ds(zb, z_len)], acc.at[pl.ds(zb, z_len)])
        pltpu.sync_copy(dst3.at[wid], didx)
        plsc.subcore_barrier()

        def s_start(j, sem):
            pltpu.async_copy(ones, acc.at[didx.at[j]], sem, add=True)

        def s_wait(sem):
            pltpu.make_async_copy(ones, acc.at[didx.at[0]], sem).wait()

        s_start(0, ssa)
        s_start(1, ssb)

        def body(jj, carry):
            s_wait(ssa)
            s_start(2 * jj + 2, ssa)
            s_wait(ssb)
            s_start(2 * jj + 3, ssb)
            return carry

        lax.fori_loop(0, (nch - 2) // 2, body, 0)
        if (nch - 2) % 2:
            s_wait(ssa)
            s_start(nch - 1, ssa)
        s_wait(ssa)
        s_wait(ssb)
        plsc.subcore_barrier()
        pltpu.sync_copy(acc.at[pl.ds(zb, o_len)],
                        out_hbm.at[c, pl.ds(zb, o_len)])

    return k


def kernel(x, edge_index, pos_encoding, params):
    n, _ = x.shape
    hd = params['W0'].shape[1]
    e = edge_index.shape[1]
    info = plsc.get_sparse_core_info()
    nc, ns = info.num_cores, info.num_subcores

    nw = nc * ns
    src = edge_index[0].astype(jnp.int32)
    dst = edge_index[1].astype(jnp.int32)
    chunk = _pick_chunk(e // nw) if e % nw == 0 else None
    if chunk is not None:
        per_tile = e // nw
        nch = per_tile // chunk
        src1, dst1 = src, dst
    else:
        chunk = 128
        per_tile = -(-e // (nw * chunk)) * chunk
        nch = per_tile // chunk
        e_pad = per_tile * nw
        src1 = jnp.concatenate(
            [src, jnp.zeros((e_pad - e,), jnp.int32)]
        ).reshape(per_tile, nw).T.reshape(-1)
        dst1 = jnp.concatenate(
            [dst, jnp.full((e_pad - e,), n, jnp.int32)]
        ).reshape(per_tile, nw).T.reshape(-1)
    dst3 = dst1.reshape(nw, nch, chunk)
    zeros2 = jnp.zeros((n + 8, hd), jnp.float32)
    ones_c = jnp.ones((chunk, hd), jnp.float32)

    deg_p = _make_sc_deg(n, hd, nch, chunk, nc, ns)(dst3, zeros2, ones_c)
    d0 = deg_p[0, :, 0].reshape(n, 1)
    d1 = deg_p[1, :, 0].reshape(n, 1)

    agg_fn = _make_sc_agg(n, hd, nch, chunk, nc, ns)
    lp = params['layers']
    hin, cs = _tc_stage_a0(x, pos_encoding, params['W0'], params['b0'],
                           lp[0]['Wc'], lp[0]['Ws'], lp[0]['bk'],
                           lp[0]['Wpe'])
    for i in range(len(lp) - 1):
        p, q = lp[i], lp[i + 1]
        agg_p = agg_fn(hin, src1, dst1, zeros2)
        hin, cs = _tc_stage_ba(hin, agg_p[0], agg_p[1], d0, d1, cs,
                               p['Wl'], p['bl'], p['Wg'], pos_encoding,
                               q['Wc'], q['Ws'], q['bk'], q['Wpe'])
    p = lp[-1]
    agg_p = agg_fn(hin, src1, dst1, zeros2)
    return _tc_stage_bf(hin, agg_p[0], agg_p[1], d0, d1, cs, p['Wl'],
                        p['bl'], p['Wg'], params['Wf'], params['bf'])


# speedup vs baseline: 2.9509x; 1.2960x over previous
import functools

import jax
import jax.numpy as jnp
from jax import lax
from jax.experimental import pallas as pl
from jax.experimental.pallas import tpu as pltpu
from jax.experimental.pallas import tpu_sc as plsc

RB = 1000


def _kan_pe(h, pe_ref, wc_ref, ws_ref, bk_ref, wpe_ref):
    return (
        jnp.dot(jnp.cos(h), wc_ref[...], preferred_element_type=jnp.float32)
        + jnp.dot(jnp.sin(h), ws_ref[...], preferred_element_type=jnp.float32)
        + bk_ref[...]
        + jnp.dot(pe_ref[...], wpe_ref[...], preferred_element_type=jnp.float32)
    )


def _combine(inv_n, hin_ref, a0_ref, a1_ref, d0_ref, d1_ref, cs_ref,
             wl_ref, bl_ref, wg_ref):
    deg = jnp.maximum(d0_ref[...] + d1_ref[...], 1.0)
    agg = (a0_ref[...] + a1_ref[...]) / deg
    local = (
        jnp.dot(agg, wl_ref[...], preferred_element_type=jnp.float32)
        + bl_ref[...]
    )
    glob = jnp.dot(cs_ref[...] * inv_n, wg_ref[...],
                   preferred_element_type=jnp.float32)
    return jnp.maximum(hin_ref[...] + local + glob, 0.0)


def _accum_colsum(hin, cs_ref):
    @pl.when(pl.program_id(0) == 0)
    def _():
        cs_ref[...] = jnp.zeros_like(cs_ref)

    cs_ref[...] += jnp.sum(hin, axis=0, keepdims=True)


def _stage_a0_body(x_ref, pe_ref, w0_ref, b0_ref, wc_ref, ws_ref, bk_ref,
                   wpe_ref, hin_ref, cs_ref):
    h = (jnp.dot(x_ref[...], w0_ref[...], preferred_element_type=jnp.float32)
         + b0_ref[...])
    hin = _kan_pe(h, pe_ref, wc_ref, ws_ref, bk_ref, wpe_ref)
    hin_ref[...] = hin
    _accum_colsum(hin, cs_ref)


def _tc_stage_a0(x, pe, w0, b0, wc, ws, bk, wpe):
    n, din = x.shape
    hd = w0.shape[1]
    p = pe.shape[1]
    full = lambda a, b: pl.BlockSpec((a, b), lambda i: (0, 0))
    return pl.pallas_call(
        _stage_a0_body,
        grid=(n // RB,),
        in_specs=[
            pl.BlockSpec((RB, din), lambda i: (i, 0)),
            pl.BlockSpec((RB, p), lambda i: (i, 0)),
            full(din, hd), full(1, hd), full(hd, hd), full(hd, hd),
            full(1, hd), full(p, hd),
        ],
        out_specs=[
            pl.BlockSpec((RB, hd), lambda i: (i, 0)),
            pl.BlockSpec((1, hd), lambda i: (0, 0)),
        ],
        out_shape=[
            jax.ShapeDtypeStruct((n, hd), jnp.float32),
            jax.ShapeDtypeStruct((1, hd), jnp.float32),
        ],
    )(x, pe, w0, b0.reshape(1, hd), wc, ws, bk.reshape(1, hd), wpe)


def _stage_ba_body(inv_n, hin_ref, a0_ref, a1_ref, d0_ref, d1_ref, cs_ref,
                   wl_ref, bl_ref, wg_ref, pe_ref, wc_ref, ws_ref, bk_ref,
                   wpe_ref, hin2_ref, cs2_ref):
    h = _combine(inv_n, hin_ref, a0_ref, a1_ref, d0_ref, d1_ref, cs_ref,
                 wl_ref, bl_ref, wg_ref)
    hin2 = _kan_pe(h, pe_ref, wc_ref, ws_ref, bk_ref, wpe_ref)
    hin2_ref[...] = hin2
    _accum_colsum(hin2, cs2_ref)


def _tc_stage_ba(hin, a0, a1, d0, d1, cs, wl, bl, wg, pe, wc, ws, bk, wpe):
    n, hd = hin.shape
    p = pe.shape[1]
    row = lambda b: pl.BlockSpec((RB, b), lambda i: (i, 0))
    full = lambda a, b: pl.BlockSpec((a, b), lambda i: (0, 0))
    return pl.pallas_call(
        functools.partial(_stage_ba_body, 1.0 / n),
        grid=(n // RB,),
        in_specs=[
            row(hd), row(hd), row(hd), row(1), row(1), full(1, hd),
            full(hd, hd), full(1, hd), full(hd, hd),
            row(p), full(hd, hd), full(hd, hd), full(1, hd), full(p, hd),
        ],
        out_specs=[
            pl.BlockSpec((RB, hd), lambda i: (i, 0)),
            pl.BlockSpec((1, hd), lambda i: (0, 0)),
        ],
        out_shape=[
            jax.ShapeDtypeStruct((n, hd), jnp.float32),
            jax.ShapeDtypeStruct((1, hd), jnp.float32),
        ],
    )(hin, a0, a1, d0, d1, cs, wl, bl.reshape(1, hd), wg,
      pe, wc, ws, bk.reshape(1, hd), wpe)


def _stage_bf_body(inv_n, hin_ref, a0_ref, a1_ref, d0_ref, d1_ref, cs_ref,
                   wl_ref, bl_ref, wg_ref, wf_ref, bf_ref, o_ref):
    h = _combine(inv_n, hin_ref, a0_ref, a1_ref, d0_ref, d1_ref, cs_ref,
                 wl_ref, bl_ref, wg_ref)
    o_ref[...] = (
        jnp.dot(h, wf_ref[...], preferred_element_type=jnp.float32)
        + bf_ref[...]
    )


def _tc_stage_bf(hin, a0, a1, d0, d1, cs, wl, bl, wg, wf, bf):
    n, hd = hin.shape
    dout = wf.shape[1]
    row = lambda b: pl.BlockSpec((RB, b), lambda i: (i, 0))
    full = lambda a, b: pl.BlockSpec((a, b), lambda i: (0, 0))
    return pl.pallas_call(
        functools.partial(_stage_bf_body, 1.0 / n),
        grid=(n // RB,),
        in_specs=[
            row(hd), row(hd), row(hd), row(1), row(1), full(1, hd),
            full(hd, hd), full(1, hd), full(hd, hd),
            full(hd, dout), full(1, dout),
        ],
        out_specs=pl.BlockSpec((RB, dout), lambda i: (i, 0)),
        out_shape=jax.ShapeDtypeStruct((n, dout), jnp.float32),
    )(hin, a0, a1, d0, d1, cs, wl, bl.reshape(1, hd), wg,
      wf, bf.reshape(1, dout))


def _pick_chunk(per_tile):
    for c in range(128, 7, -8):
        if per_tile % c == 0:
            return c
    return None


def _make_sc_agg(n, hd, nch, chunk, nc, ns):
    zb_step = (n // ns) // 8 * 8
    z_len = (n + 8) - (ns - 1) * zb_step
    o_len = n - (ns - 1) * zb_step
    mesh = plsc.VectorSubcoreMesh(core_axis_name="c", subcore_axis_name="s")

    @functools.partial(
        pl.kernel,
        mesh=mesh,
        out_type=jax.ShapeDtypeStruct((nc, n, hd), jnp.float32),
        scratch_types=[
            pltpu.VMEM((chunk,), jnp.int32),
            pltpu.VMEM((chunk,), jnp.int32),
            pltpu.VMEM((chunk,), jnp.int32),
            pltpu.VMEM((chunk,), jnp.int32),
            pltpu.VMEM((chunk, hd), jnp.float32),
            pltpu.VMEM((chunk, hd), jnp.float32),
            pltpu.SemaphoreType.DMA,
            pltpu.SemaphoreType.DMA,
            pltpu.SemaphoreType.DMA,
            pltpu.VMEM_SHARED((n + 8, hd), jnp.float32),
        ],
    )
    def k(h_hbm, src1, dst1, z_hbm, out_hbm, sidx_a, didx_a, sidx_b, didx_b,
          rows_a, rows_b, sg, ssa, ssb, acc):
        c = lax.axis_index("c")
        s = lax.axis_index("s")
        wid = c * ns + s
        per_tile = nch * chunk
        zb = s * zb_step
        pltpu.sync_copy(z_hbm.at[pl.ds(zb, z_len)], acc.at[pl.ds(zb, z_len)])
        plsc.subcore_barrier()

        seta = (sidx_a, didx_a, rows_a, ssa)
        setb = (sidx_b, didx_b, rows_b, ssb)

        def copy_idx(j, st):
            base = wid * per_tile + j * chunk
            pltpu.sync_copy(src1.at[pl.ds(base, chunk)], st[0])
            pltpu.sync_copy(dst1.at[pl.ds(base, chunk)], st[1])

        def g_start(st):
            pltpu.async_copy(h_hbm.at[st[0]], st[2], sg)

        def g_wait(st):
            pltpu.make_async_copy(h_hbm.at[st[0]], st[2], sg).wait()

        def s_start(st):
            pltpu.async_copy(st[2], acc.at[st[1]], st[3], add=True)

        def s_wait(st):
            pltpu.make_async_copy(st[2], acc.at[st[1]], st[3]).wait()

        def stage(j, cur, nxt, first, has_next):
            if not first:
                s_wait(nxt)
            if has_next:
                copy_idx(j + 1, nxt)
            g_wait(cur)
            if has_next:
                g_start(nxt)
            s_start(cur)

        copy_idx(0, seta)
        g_start(seta)
        stage(0, seta, setb, True, nch > 1)
        npairs = max(0, (nch - 2) // 2)

        def body(k, carry):
            stage(2 * k + 1, setb, seta, False, True)
            stage(2 * k + 2, seta, setb, False, True)
            return carry

        lax.fori_loop(0, npairs, body, 0)
        for j in range(2 * npairs + 1, nch):
            cur, nxt = (seta, setb) if j % 2 == 0 else (setb, seta)
            stage(j, cur, nxt, False, j + 1 < nch)
        s_wait(seta if (nch - 1) % 2 == 0 else setb)
        plsc.subcore_barrier()
        pltpu.sync_copy(acc.at[pl.ds(zb, o_len)],
                        out_hbm.at[c, pl.ds(zb, o_len)])

    return k


def _make_sc_deg(n, hd, nch, chunk, nc, ns):
    zb_step = (n // ns) // 8 * 8
    z_len = (n + 8) - (ns - 1) * zb_step
    o_len = n - (ns - 1) * zb_step
    mesh = plsc.VectorSubcoreMesh(core_axis_name="c", subcore_axis_name="s")

    @functools.partial(
        pl.kernel,
        mesh=mesh,
        out_type=jax.ShapeDtypeStruct((nc, n, hd), jnp.float32),
        scratch_types=[
            pltpu.VMEM((nch, chunk), jnp.int32),
            pltpu.VMEM((chunk, hd), jnp.float32),
            pltpu.SemaphoreType.DMA,
            pltpu.SemaphoreType.DMA,
            pltpu.VMEM_SHARED((n + 8, hd), jnp.float32),
        ],
    )
    def k(dst3, z_hbm, ones_hbm, out_hbm, didx, ones, ssa, ssb, acc):
        c = lax.axis_index("c")
        s = lax.axis_index("s")
        wid = c * ns + s
        zb = s * zb_step
        pltpu.sync_copy(ones_hbm, ones)
        pltpu.sync_copy(z_hbm.at[pl.ds(zb, z_len)], acc.at[pl.ds(zb, z_len)])
        pltpu.sync_copy(dst3.at[wid], didx)
        plsc.subcore_barrier()

        def s_start(j, sem):
            pltpu.async_copy(ones, acc.at[didx.at[j]], sem, add=True)

        def s_wait(sem):
            pltpu.make_async_copy(ones, acc.at[didx.at[0]], sem).wait()

        s_start(0, ssa)
        s_start(1, ssb)

        def body(jj, carry):
            s_wait(ssa)
            s_start(2 * jj + 2, ssa)
            s_wait(ssb)
            s_start(2 * jj + 3, ssb)
            return carry

        lax.fori_loop(0, (nch - 2) // 2, body, 0)
        if (nch - 2) % 2:
            s_wait(ssa)
            s_start(nch - 1, ssa)
        s_wait(ssa)
        s_wait(ssb)
        plsc.subcore_barrier()
        pltpu.sync_copy(acc.at[pl.ds(zb, o_len)],
                        out_hbm.at[c, pl.ds(zb, o_len)])

    return k


def kernel(x, edge_index, pos_encoding, params):
    n, _ = x.shape
    hd = params['W0'].shape[1]
    e = edge_index.shape[1]
    info = plsc.get_sparse_core_info()
    nc, ns = info.num_cores, info.num_subcores

    nw = nc * ns
    src = edge_index[0].astype(jnp.int32)
    dst = edge_index[1].astype(jnp.int32)
    chunk = _pick_chunk(e // nw) if e % nw == 0 else None
    if chunk is not None:
        per_tile = e // nw
        nch = per_tile // chunk
        src1, dst1 = src, dst
    else:
        chunk = 128
        per_tile = -(-e // (nw * chunk)) * chunk
        nch = per_tile // chunk
        e_pad = per_tile * nw
        src1 = jnp.concatenate(
            [src, jnp.zeros((e_pad - e,), jnp.int32)]
        ).reshape(per_tile, nw).T.reshape(-1)
        dst1 = jnp.concatenate(
            [dst, jnp.full((e_pad - e,), n, jnp.int32)]
        ).reshape(per_tile, nw).T.reshape(-1)
    dst3 = dst1.reshape(nw, nch, chunk)
    zeros2 = jnp.zeros((n + 8, hd), jnp.float32)
    ones_c = jnp.ones((chunk, hd), jnp.float32)

    deg_p = _make_sc_deg(n, hd, nch, chunk, nc, ns)(dst3, zeros2, ones_c)
    d0 = deg_p[0, :, 0].reshape(n, 1)
    d1 = deg_p[1, :, 0].reshape(n, 1)

    agg_fn = _make_sc_agg(n, hd, nch, chunk, nc, ns)
    lp = params['layers']
    hin, cs = _tc_stage_a0(x, pos_encoding, params['W0'], params['b0'],
                           lp[0]['Wc'], lp[0]['Ws'], lp[0]['bk'],
                           lp[0]['Wpe'])
    for i in range(len(lp) - 1):
        p, q = lp[i], lp[i + 1]
        agg_p = agg_fn(hin, src1, dst1, zeros2)
        hin, cs = _tc_stage_ba(hin, agg_p[0], agg_p[1], d0, d1, cs,
                               p['Wl'], p['bl'], p['Wg'], pos_encoding,
                               q['Wc'], q['Ws'], q['bk'], q['Wpe'])
    p = lp[-1]
    agg_p = agg_fn(hin, src1, dst1, zeros2)
    return _tc_stage_bf(hin, agg_p[0], agg_p[1], d0, d1, cs, p['Wl'],
                        p['bl'], p['Wg'], params['Wf'], params['bf'])


# speedup vs baseline: 2.9525x; 1.0005x over previous
import functools

import jax
import jax.numpy as jnp
from jax import lax
from jax.experimental import pallas as pl
from jax.experimental.pallas import tpu as pltpu
from jax.experimental.pallas import tpu_sc as plsc

RB = 1000


def _kan_pe(h, pe_ref, wc_ref, ws_ref, bk_ref, wpe_ref):
    return (
        jnp.dot(jnp.cos(h), wc_ref[...], preferred_element_type=jnp.float32)
        + jnp.dot(jnp.sin(h), ws_ref[...], preferred_element_type=jnp.float32)
        + bk_ref[...]
        + jnp.dot(pe_ref[...], wpe_ref[...], preferred_element_type=jnp.float32)
    )


def _combine(inv_n, hin_ref, a0_ref, a1_ref, d0_ref, d1_ref, cs_ref,
             wl_ref, bl_ref, wg_ref):
    deg = jnp.maximum(d0_ref[...] + d1_ref[...], 1.0)
    agg = (a0_ref[...] + a1_ref[...]) / deg
    local = (
        jnp.dot(agg, wl_ref[...], preferred_element_type=jnp.float32)
        + bl_ref[...]
    )
    glob = jnp.dot(cs_ref[...] * inv_n, wg_ref[...],
                   preferred_element_type=jnp.float32)
    return jnp.maximum(hin_ref[...] + local + glob, 0.0)


def _accum_colsum(hin, cs_ref):
    @pl.when(pl.program_id(0) == 0)
    def _():
        cs_ref[...] = jnp.zeros_like(cs_ref)

    cs_ref[...] += jnp.sum(hin, axis=0, keepdims=True)


def _stage_a0_body(x_ref, pe_ref, w0_ref, b0_ref, wc_ref, ws_ref, bk_ref,
                   wpe_ref, hin_ref, cs_ref):
    h = (jnp.dot(x_ref[...], w0_ref[...], preferred_element_type=jnp.float32)
         + b0_ref[...])
    hin = _kan_pe(h, pe_ref, wc_ref, ws_ref, bk_ref, wpe_ref)
    hin_ref[...] = hin
    _accum_colsum(hin, cs_ref)


def _tc_stage_a0(x, pe, w0, b0, wc, ws, bk, wpe):
    n, din = x.shape
    hd = w0.shape[1]
    p = pe.shape[1]
    full = lambda a, b: pl.BlockSpec((a, b), lambda i: (0, 0))
    return pl.pallas_call(
        _stage_a0_body,
        grid=(n // RB,),
        in_specs=[
            pl.BlockSpec((RB, din), lambda i: (i, 0)),
            pl.BlockSpec((RB, p), lambda i: (i, 0)),
            full(din, hd), full(1, hd), full(hd, hd), full(hd, hd),
            full(1, hd), full(p, hd),
        ],
        out_specs=[
            pl.BlockSpec((RB, hd), lambda i: (i, 0)),
            pl.BlockSpec((1, hd), lambda i: (0, 0)),
        ],
        out_shape=[
            jax.ShapeDtypeStruct((n, hd), jnp.float32),
            jax.ShapeDtypeStruct((1, hd), jnp.float32),
        ],
    )(x, pe, w0, b0.reshape(1, hd), wc, ws, bk.reshape(1, hd), wpe)


def _stage_ba_body(inv_n, hin_ref, a0_ref, a1_ref, d0_ref, d1_ref, cs_ref,
                   wl_ref, bl_ref, wg_ref, pe_ref, wc_ref, ws_ref, bk_ref,
                   wpe_ref, hin2_ref, cs2_ref):
    h = _combine(inv_n, hin_ref, a0_ref, a1_ref, d0_ref, d1_ref, cs_ref,
                 wl_ref, bl_ref, wg_ref)
    hin2 = _kan_pe(h, pe_ref, wc_ref, ws_ref, bk_ref, wpe_ref)
    hin2_ref[...] = hin2
    _accum_colsum(hin2, cs2_ref)


def _tc_stage_ba(hin, a0, a1, d0, d1, cs, wl, bl, wg, pe, wc, ws, bk, wpe):
    n, hd = hin.shape
    p = pe.shape[1]
    row = lambda b: pl.BlockSpec((RB, b), lambda i: (i, 0))
    full = lambda a, b: pl.BlockSpec((a, b), lambda i: (0, 0))
    return pl.pallas_call(
        functools.partial(_stage_ba_body, 1.0 / n),
        grid=(n // RB,),
        in_specs=[
            row(hd), row(hd), row(hd), row(1), row(1), full(1, hd),
            full(hd, hd), full(1, hd), full(hd, hd),
            row(p), full(hd, hd), full(hd, hd), full(1, hd), full(p, hd),
        ],
        out_specs=[
            pl.BlockSpec((RB, hd), lambda i: (i, 0)),
            pl.BlockSpec((1, hd), lambda i: (0, 0)),
        ],
        out_shape=[
            jax.ShapeDtypeStruct((n, hd), jnp.float32),
            jax.ShapeDtypeStruct((1, hd), jnp.float32),
        ],
    )(hin, a0, a1, d0, d1, cs, wl, bl.reshape(1, hd), wg,
      pe, wc, ws, bk.reshape(1, hd), wpe)


def _stage_bf_body(inv_n, hin_ref, a0_ref, a1_ref, d0_ref, d1_ref, cs_ref,
                   wl_ref, bl_ref, wg_ref, wf_ref, bf_ref, o_ref):
    h = _combine(inv_n, hin_ref, a0_ref, a1_ref, d0_ref, d1_ref, cs_ref,
                 wl_ref, bl_ref, wg_ref)
    o_ref[...] = (
        jnp.dot(h, wf_ref[...], preferred_element_type=jnp.float32)
        + bf_ref[...]
    )


def _tc_stage_bf(hin, a0, a1, d0, d1, cs, wl, bl, wg, wf, bf):
    n, hd = hin.shape
    dout = wf.shape[1]
    row = lambda b: pl.BlockSpec((RB, b), lambda i: (i, 0))
    full = lambda a, b: pl.BlockSpec((a, b), lambda i: (0, 0))
    return pl.pallas_call(
        functools.partial(_stage_bf_body, 1.0 / n),
        grid=(n // RB,),
        in_specs=[
            row(hd), row(hd), row(hd), row(1), row(1), full(1, hd),
            full(hd, hd), full(1, hd), full(hd, hd),
            full(hd, dout), full(1, dout),
        ],
        out_specs=pl.BlockSpec((RB, dout), lambda i: (i, 0)),
        out_shape=jax.ShapeDtypeStruct((n, dout), jnp.float32),
    )(hin, a0, a1, d0, d1, cs, wl, bl.reshape(1, hd), wg,
      wf, bf.reshape(1, dout))


def _pick_chunk(per_tile):
    for c in range(128, 7, -8):
        if per_tile % c == 0:
            return c
    return None


def _make_sc_agg(n, hd, nch, chunk, nc, ns):
    zb_step = (n // ns) // 8 * 8
    z_len = (n + 8) - (ns - 1) * zb_step
    o_len = n - (ns - 1) * zb_step
    mesh = plsc.VectorSubcoreMesh(core_axis_name="c", subcore_axis_name="s")

    @functools.partial(
        pl.kernel,
        mesh=mesh,
        out_type=jax.ShapeDtypeStruct((nc, n, hd), jnp.float32),
        scratch_types=[
            pltpu.VMEM((chunk,), jnp.int32),
            pltpu.VMEM((chunk,), jnp.int32),
            pltpu.VMEM((chunk,), jnp.int32),
            pltpu.VMEM((chunk,), jnp.int32),
            pltpu.VMEM((chunk, hd), jnp.float32),
            pltpu.VMEM((chunk, hd), jnp.float32),
            pltpu.SemaphoreType.DMA,
            pltpu.SemaphoreType.DMA,
            pltpu.SemaphoreType.DMA,
            pltpu.SemaphoreType.DMA,
            pltpu.VMEM_SHARED((n + 8, hd), jnp.float32),
        ],
    )
    def k(h_hbm, src1, dst1, z_hbm, out_hbm, sidx_a, didx_a, sidx_b, didx_b,
          rows_a, rows_b, sga, sgb, ssa, ssb, acc):
        c = lax.axis_index("c")
        s = lax.axis_index("s")
        wid = c * ns + s
        per_tile = nch * chunk
        zb = s * zb_step
        pltpu.sync_copy(z_hbm.at[pl.ds(zb, z_len)], acc.at[pl.ds(zb, z_len)])
        plsc.subcore_barrier()

        seta = (sidx_a, didx_a, rows_a, ssa, sga)
        setb = (sidx_b, didx_b, rows_b, ssb, sgb)

        def copy_idx(j, st):
            base = wid * per_tile + j * chunk
            pltpu.sync_copy(src1.at[pl.ds(base, chunk)], st[0])
            pltpu.sync_copy(dst1.at[pl.ds(base, chunk)], st[1])

        def g_start(st):
            pltpu.async_copy(h_hbm.at[st[0]], st[2], st[4])

        def g_wait(st):
            pltpu.make_async_copy(h_hbm.at[st[0]], st[2], st[4]).wait()

        def s_start(st):
            pltpu.async_copy(st[2], acc.at[st[1]], st[3], add=True)

        def s_wait(st):
            pltpu.make_async_copy(st[2], acc.at[st[1]], st[3]).wait()

        def stage(j, cur, nxt, first, has_next):
            if not first:
                s_wait(nxt)
            if has_next:
                copy_idx(j + 1, nxt)
                g_start(nxt)
            g_wait(cur)
            s_start(cur)

        copy_idx(0, seta)
        g_start(seta)
        stage(0, seta, setb, True, nch > 1)
        npairs = max(0, (nch - 2) // 2)

        def body(k, carry):
            stage(2 * k + 1, setb, seta, False, True)
            stage(2 * k + 2, seta, setb, False, True)
            return carry

        lax.fori_loop(0, npairs, body, 0)
        for j in range(2 * npairs + 1, nch):
            cur, nxt = (seta, setb) if j % 2 == 0 else (setb, seta)
            stage(j, cur, nxt, False, j + 1 < nch)
        s_wait(seta if (nch - 1) % 2 == 0 else setb)
        plsc.subcore_barrier()
        pltpu.sync_copy(acc.at[pl.ds(zb, o_len)],
                        out_hbm.at[c, pl.ds(zb, o_len)])

    return k


def _make_sc_deg(n, hd, nch, chunk, nc, ns):
    zb_step = (n // ns) // 8 * 8
    z_len = (n + 8) - (ns - 1) * zb_step
    o_len = n - (ns - 1) * zb_step
    mesh = plsc.VectorSubcoreMesh(core_axis_name="c", subcore_axis_name="s")

    @functools.partial(
        pl.kernel,
        mesh=mesh,
        out_type=jax.ShapeDtypeStruct((nc, n, hd), jnp.float32),
        scratch_types=[
            pltpu.VMEM((nch, chunk), jnp.int32),
            pltpu.VMEM((chunk, hd), jnp.float32),
            pltpu.SemaphoreType.DMA,
            pltpu.SemaphoreType.DMA,
            pltpu.VMEM_SHARED((n + 8, hd), jnp.float32),
        ],
    )
    def k(dst3, z_hbm, ones_hbm, out_hbm, didx, ones, ssa, ssb, acc):
        c = lax.axis_index("c")
        s = lax.axis_index("s")
        wid = c * ns + s
        zb = s * zb_step
        pltpu.sync_copy(ones_hbm, ones)
        pltpu.sync_copy(z_hbm.at[pl.ds(zb, z_len)], acc.at[pl.ds(zb, z_len)])
        pltpu.sync_copy(dst3.at[wid], didx)
        plsc.subcore_barrier()

        def s_start(j, sem):
            pltpu.async_copy(ones, acc.at[didx.at[j]], sem, add=True)

        def s_wait(sem):
            pltpu.make_async_copy(ones, acc.at[didx.at[0]], sem).wait()

        s_start(0, ssa)
        s_start(1, ssb)

        def body(jj, carry):
            s_wait(ssa)
            s_start(2 * jj + 2, ssa)
            s_wait(ssb)
            s_start(2 * jj + 3, ssb)
            return carry

        lax.fori_loop(0, (nch - 2) // 2, body, 0)
        if (nch - 2) % 2:
            s_wait(ssa)
            s_start(nch - 1, ssa)
        s_wait(ssa)
        s_wait(ssb)
        plsc.subcore_barrier()
        pltpu.sync_copy(acc.at[pl.ds(zb, o_len)],
                        out_hbm.at[c, pl.ds(zb, o_len)])

    return k


def kernel(x, edge_index, pos_encoding, params):
    n, _ = x.shape
    hd = params['W0'].shape[1]
    e = edge_index.shape[1]
    info = plsc.get_sparse_core_info()
    nc, ns = info.num_cores, info.num_subcores

    nw = nc * ns
    src = edge_index[0].astype(jnp.int32)
    dst = edge_index[1].astype(jnp.int32)
    chunk = _pick_chunk(e // nw) if e % nw == 0 else None
    if chunk is not None:
        per_tile = e // nw
        nch = per_tile // chunk
        src1, dst1 = src, dst
    else:
        chunk = 128
        per_tile = -(-e // (nw * chunk)) * chunk
        nch = per_tile // chunk
        e_pad = per_tile * nw
        src1 = jnp.concatenate(
            [src, jnp.zeros((e_pad - e,), jnp.int32)]
        ).reshape(per_tile, nw).T.reshape(-1)
        dst1 = jnp.concatenate(
            [dst, jnp.full((e_pad - e,), n, jnp.int32)]
        ).reshape(per_tile, nw).T.reshape(-1)
    dst3 = dst1.reshape(nw, nch, chunk)
    zeros2 = jnp.zeros((n + 8, hd), jnp.float32)
    ones_c = jnp.ones((chunk, hd), jnp.float32)

    deg_p = _make_sc_deg(n, hd, nch, chunk, nc, ns)(dst3, zeros2, ones_c)
    d0 = deg_p[0, :, 0].reshape(n, 1)
    d1 = deg_p[1, :, 0].reshape(n, 1)

    agg_fn = _make_sc_agg(n, hd, nch, chunk, nc, ns)
    lp = params['layers']
    hin, cs = _tc_stage_a0(x, pos_encoding, params['W0'], params['b0'],
                           lp[0]['Wc'], lp[0]['Ws'], lp[0]['bk'],
                           lp[0]['Wpe'])
    for i in range(len(lp) - 1):
        p, q = lp[i], lp[i + 1]
        agg_p = agg_fn(hin, src1, dst1, zeros2)
        hin, cs = _tc_stage_ba(hin, agg_p[0], agg_p[1], d0, d1, cs,
                               p['Wl'], p['bl'], p['Wg'], pos_encoding,
                               q['Wc'], q['Ws'], q['bk'], q['Wpe'])
    p = lp[-1]
    agg_p = agg_fn(hin, src1, dst1, zeros2)
    return _tc_stage_bf(hin, agg_p[0], agg_p[1], d0, d1, cs, p['Wl'],
                        p['bl'], p['Wg'], params['Wf'], params['bf'])
